# baseline (speedup 1.0000x reference)
import functools

import jax
import jax.numpy as jnp
from jax import lax
from jax.experimental import pallas as pl
from jax.experimental.pallas import tpu as pltpu
from jax.experimental.pallas import tpu_sc as plsc

_B, _L, _V, _P, _T, _D = 1024, 512, 100000, 512, 2, 128
_N = _B * _L
_EPS = 1e-12

_NW = 32
_TOK_W = _N // _NW
_C = 256
_NCH = _TOK_W // _C
_LANES = 16
_DC = _D // _LANES


def _rsqrt(v):
    i = lax.bitcast_convert_type(v, jnp.int32)
    i = jnp.int32(0x5F3759DF) - lax.shift_right_logical(i, 1)
    y = lax.bitcast_convert_type(i, jnp.float32)
    for _ in range(3):
        y = y * (jnp.float32(1.5) - jnp.float32(0.5) * v * y * y)
    return y


def _body(ids_hbm, tt_hbm, wemb_hbm, pemb_t_hbm, temb_hbm, gam_hbm, bet_hbm,
          out_hbm, idx_v, tt_v, rows_v, pos_t_v, typ_v, gam_v, bet_v,
          a_v, b_v, sem):
    c = lax.axis_index("c")
    s = lax.axis_index("s")
    wid = s * 2 + c
    base_w = wid * _TOK_W

    pltpu.sync_copy(pemb_t_hbm, pos_t_v)
    pltpu.sync_copy(temb_hbm, typ_v)
    pltpu.sync_copy(gam_hbm, gam_v)
    pltpu.sync_copy(bet_hbm, bet_v)

    def chunk_body(g, carry):
        base = base_w + g * _C
        gm = lax.rem(g, 4)

        @pl.when(gm == 0)
        def _():
            row0 = pl.multiple_of(base // 128, 8)
            pltpu.sync_copy(ids_hbm.at[pl.ds(row0, 8)], idx_v)

        pltpu.sync_copy(tt_hbm.at[pl.ds(base, _C)], tt_v)
        cps = [
            pltpu.async_copy(
                wemb_hbm.at[idx_v.at[gm * 2 + j]],
                rows_v.at[pl.ds(j * 128, 128)],
                sem,
            )
            for j in range(_C // 128)
        ]
        for cp in cps:
            cp.wait()

        pw = (g % 2) * _C
        rd = jnp.float32(1.0 / _D)
        zer = jnp.zeros((_LANES,), jnp.float32)
        izer = jnp.zeros((_LANES,), jnp.int32)

        def group_body(g16, tc):
            i0 = g16 * _LANES
            rows16 = i0 + lax.iota(jnp.int32, _LANES)
            tt16 = tt_v[pl.ds(i0, _LANES)]

            def d_body(d, accs):
                acc_s, acc_q = accs
                dspl = izer + d
                x = plsc.load_gather(rows_v, [rows16, dspl])
                p = pos_t_v[d, pl.ds(pw + i0, _LANES)]
                t = plsc.load_gather(typ_v, [tt16, dspl])
                x = x + p + t
                plsc.store_scatter(rows_v, [rows16, dspl], x)
                return (acc_s + x, acc_q + x * x)

            acc_s, acc_q = lax.fori_loop(0, _D, d_body, (zer, zer),
                                         unroll=4)
            mean = acc_s * rd
            var = acc_q * rd - mean * mean
            inv = _rsqrt(var + jnp.float32(_EPS))
            a_v[pl.ds(i0, _LANES)] = inv
            b_v[pl.ds(i0, _LANES)] = mean * inv
            return tc

        lax.fori_loop(0, _C // _LANES, group_body, 0)

        gs = [gam_v[pl.ds(cc * _LANES, _LANES)] for cc in range(_DC)]
        bs = [bet_v[pl.ds(cc * _LANES, _LANES)] for cc in range(_DC)]

        def norm_body(i, tc):
            ispl = izer + i
            av = plsc.load_gather(a_v, [ispl])
            bv = plsc.load_gather(b_v, [ispl])
            for cc in range(_DC):
                sl = pl.ds(cc * _LANES, _LANES)
                x = rows_v[i, sl]
                rows_v[i, sl] = (x * av - bv) * gs[cc] + bs[cc]
            return tc

        lax.fori_loop(0, _C, norm_body, 0, unroll=2)
        pltpu.sync_copy(rows_v, out_hbm.at[pl.ds(base, _C)])
        return carry

    lax.fori_loop(0, _NCH, chunk_body, 0, unroll=False)


_sc_call = pl.kernel(
    _body,
    out_type=jax.ShapeDtypeStruct((_N, _D), jnp.float32),
    mesh=plsc.VectorSubcoreMesh(core_axis_name="c", subcore_axis_name="s"),
    compiler_params=pltpu.CompilerParams(needs_layout_passes=False),
    scratch_types=[
        pltpu.VMEM((8, 128), jnp.int32),
        pltpu.VMEM((_C,), jnp.int32),
        pltpu.VMEM((_C, _D), jnp.float32),
        pltpu.VMEM((_D, _P), jnp.float32),
        pltpu.VMEM((_T, _D), jnp.float32),
        pltpu.VMEM((_D,), jnp.float32),
        pltpu.VMEM((_D,), jnp.float32),
        pltpu.VMEM((_C,), jnp.float32),
        pltpu.VMEM((_C,), jnp.float32),
        pltpu.SemaphoreType.DMA,
    ],
)


def kernel(input_ids, token_type_ids, word_emb, pos_emb, type_emb,
           ln_gamma, ln_beta):
    ids = input_ids.reshape(-1).astype(jnp.int32).reshape(_N // 128, 128)
    tt = token_type_ids.reshape(-1).astype(jnp.int32)
    out = _sc_call(ids, tt, word_emb.astype(jnp.float32),
                   pos_emb.astype(jnp.float32).T,
                   type_emb.astype(jnp.float32),
                   ln_gamma.astype(jnp.float32), ln_beta.astype(jnp.float32))
    return out.reshape(_B, _L, _D)


# speedup vs baseline: 3.8415x; 3.8415x over previous
import functools

import jax
import jax.numpy as jnp
from jax import lax
from jax.experimental import pallas as pl
from jax.experimental.pallas import tpu as pltpu
from jax.experimental.pallas import tpu_sc as plsc

_B, _L, _V, _P, _T, _D = 1024, 512, 100000, 512, 2, 128
_N = _B * _L
_EPS = 1e-12

_NW = 32
_TOK_W = _N // _NW
_C = 256
_NCH = _TOK_W // _C


def _gather_body(ids_hbm, wemb_hbm, out_hbm, idx_v, rows_a, rows_b, sem):
    c = lax.axis_index("c")
    s = lax.axis_index("s")
    wid = s * 2 + c
    base_w = wid * _TOK_W

    def chunk_body(g, carry):
        base = base_w + g * _C
        gm = lax.rem(g, 4)

        @pl.when(gm == 0)
        def _():
            row0 = pl.multiple_of(base // 128, 8)
            pltpu.sync_copy(ids_hbm.at[pl.ds(row0, 8)], idx_v)

        def run(rows_v):
            cps = [
                pltpu.async_copy(
                    wemb_hbm.at[idx_v.at[gm * 2 + j]],
                    rows_v.at[pl.ds(j * 128, 128)],
                    sem,
                )
                for j in range(_C // 128)
            ]
            for cp in cps:
                cp.wait()
            pltpu.sync_copy(rows_v, out_hbm.at[pl.ds(base, _C)])

        @pl.when(lax.rem(g, 2) == 0)
        def _():
            run(rows_a)

        @pl.when(lax.rem(g, 2) == 1)
        def _():
            run(rows_b)

        return carry

    lax.fori_loop(0, _NCH, chunk_body, 0)


_sc_gather = pl.kernel(
    _gather_body,
    out_type=jax.ShapeDtypeStruct((_N, _D), jnp.float32),
    mesh=plsc.VectorSubcoreMesh(core_axis_name="c", subcore_axis_name="s"),
    compiler_params=pltpu.CompilerParams(needs_layout_passes=False),
    scratch_types=[
        pltpu.VMEM((8, 128), jnp.int32),
        pltpu.VMEM((_C, _D), jnp.float32),
        pltpu.VMEM((_C, _D), jnp.float32),
        pltpu.SemaphoreType.DMA,
    ],
)


def _ln_body(x_ref, ttf_ref, pos_ref, temb_ref, gam_ref, bet_ref, o_ref):
    x = x_ref[...]
    ttf = ttf_ref[...]
    t0 = temb_ref[0:1, :]
    t1 = temb_ref[1:2, :]
    x = x + pos_ref[...] + t0 + ttf * (t1 - t0)
    mean = jnp.mean(x, axis=-1, keepdims=True)
    xc = x - mean
    var = jnp.mean(xc * xc, axis=-1, keepdims=True)
    inv = lax.rsqrt(var + _EPS)
    o_ref[...] = xc * inv * gam_ref[...] + bet_ref[...]


_tc_ln = pl.pallas_call(
    _ln_body,
    grid=(_B,),
    in_specs=[
        pl.BlockSpec((_L, _D), lambda i: (i, 0)),
        pl.BlockSpec((_L, 1), lambda i: (i, 0)),
        pl.BlockSpec((_L, _D), lambda i: (0, 0)),
        pl.BlockSpec((_T, _D), lambda i: (0, 0)),
        pl.BlockSpec((1, _D), lambda i: (0, 0)),
        pl.BlockSpec((1, _D), lambda i: (0, 0)),
    ],
    out_specs=pl.BlockSpec((_L, _D), lambda i: (i, 0)),
    out_shape=jax.ShapeDtypeStruct((_N, _D), jnp.float32),
)


def kernel(input_ids, token_type_ids, word_emb, pos_emb, type_emb,
           ln_gamma, ln_beta):
    ids = input_ids.reshape(-1).astype(jnp.int32).reshape(_N // 128, 128)
    ttf = token_type_ids.reshape(_N, 1).astype(jnp.float32)
    rows = _sc_gather(ids, word_emb.astype(jnp.float32))
    out = _tc_ln(rows, ttf, pos_emb.astype(jnp.float32),
                 type_emb.astype(jnp.float32),
                 ln_gamma.astype(jnp.float32).reshape(1, _D),
                 ln_beta.astype(jnp.float32).reshape(1, _D))
    return out.reshape(_B, _L, _D)


# speedup vs baseline: 6.9115x; 1.7992x over previous
import functools

import jax
import jax.numpy as jnp
from jax import lax
from jax.experimental import pallas as pl
from jax.experimental.pallas import tpu as pltpu
from jax.experimental.pallas import tpu_sc as plsc

_B, _L, _V, _P, _T, _D = 1024, 512, 100000, 512, 2, 128
_N = _B * _L
_EPS = 1e-12

_NW = 32
_TOK_W = _N // _NW
_C = 256
_NCH = _TOK_W // _C


def _gather_body(ids_hbm, wemb_hbm, out_hbm, idx_v, rows_a, rows_b, sem):
    c = lax.axis_index("c")
    s = lax.axis_index("s")
    wid = s * 2 + c
    base_w = wid * _TOK_W

    def chunk_body(g, carry):
        base = base_w + g * _C
        gm = lax.rem(g, 4)

        @pl.when(gm == 0)
        def _():
            row0 = pl.multiple_of(base // 128, 8)
            pltpu.sync_copy(ids_hbm.at[pl.ds(row0, 8)], idx_v)

        def run(rows_v):
            cps = [
                pltpu.async_copy(
                    wemb_hbm.at[idx_v.at[gm * 2 + j]],
                    rows_v.at[pl.ds(j * 128, 128)],
                    sem,
                )
                for j in range(_C // 128)
            ]
            for cp in cps:
                cp.wait()
            pltpu.sync_copy(rows_v, out_hbm.at[pl.ds(base, _C)])

        @pl.when(lax.rem(g, 2) == 0)
        def _():
            run(rows_a)

        @pl.when(lax.rem(g, 2) == 1)
        def _():
            run(rows_b)

        return carry

    lax.fori_loop(0, _NCH, chunk_body, 0)


_sc_gather = pl.kernel(
    _gather_body,
    out_type=jax.ShapeDtypeStruct((_N, _D), jnp.float32),
    mesh=plsc.VectorSubcoreMesh(core_axis_name="c", subcore_axis_name="s"),
    compiler_params=pltpu.CompilerParams(needs_layout_passes=False),
    scratch_types=[
        pltpu.VMEM((8, 128), jnp.int32),
        pltpu.VMEM((_C, _D), jnp.float32),
        pltpu.VMEM((_C, _D), jnp.float32),
        pltpu.SemaphoreType.DMA,
    ],
)


_SPS = 8
_BLK = _SPS * _L


def _ln_body(x_ref, ttf_ref, pos_ref, temb_ref, gam_ref, bet_ref, o_ref):
    x = x_ref[...]
    ttf = ttf_ref[...]
    t0 = temb_ref[0:1, :]
    t1 = temb_ref[1:2, :]
    x = x + pos_ref[...] + t0 + ttf * (t1 - t0)
    mean = jnp.mean(x, axis=-1, keepdims=True)
    xc = x - mean
    var = jnp.mean(xc * xc, axis=-1, keepdims=True)
    inv = lax.rsqrt(var + _EPS)
    o_ref[...] = xc * inv * gam_ref[...] + bet_ref[...]


_tc_ln = pl.pallas_call(
    _ln_body,
    grid=(_B // _SPS,),
    in_specs=[
        pl.BlockSpec((_BLK, _D), lambda i: (i, 0)),
        pl.BlockSpec((_BLK, 1), lambda i: (i, 0)),
        pl.BlockSpec((_BLK, _D), lambda i: (0, 0)),
        pl.BlockSpec((_T, _D), lambda i: (0, 0)),
        pl.BlockSpec((1, _D), lambda i: (0, 0)),
        pl.BlockSpec((1, _D), lambda i: (0, 0)),
    ],
    out_specs=pl.BlockSpec((_BLK, _D), lambda i: (i, 0)),
    out_shape=jax.ShapeDtypeStruct((_N, _D), jnp.float32),
)


def kernel(input_ids, token_type_ids, word_emb, pos_emb, type_emb,
           ln_gamma, ln_beta):
    ids = input_ids.reshape(-1).astype(jnp.int32).reshape(_N // 128, 128)
    ttf = token_type_ids.reshape(_N, 1).astype(jnp.float32)
    rows = _sc_gather(ids, word_emb.astype(jnp.float32))
    pos_t = jnp.tile(pos_emb.astype(jnp.float32), (_SPS, 1))
    out = _tc_ln(rows, ttf, pos_t, type_emb.astype(jnp.float32),
                 ln_gamma.astype(jnp.float32).reshape(1, _D),
                 ln_beta.astype(jnp.float32).reshape(1, _D))
    return out.reshape(_B, _L, _D)


# speedup vs baseline: 6.9544x; 1.0062x over previous
import functools

import jax
import jax.numpy as jnp
from jax import lax
from jax.experimental import pallas as pl
from jax.experimental.pallas import tpu as pltpu
from jax.experimental.pallas import tpu_sc as plsc

_B, _L, _V, _P, _T, _D = 1024, 512, 100000, 512, 2, 128
_N = _B * _L
_EPS = 1e-12

_NW = 32
_TOK_W = _N // _NW
_C = 256
_NCH = _TOK_W // _C


def _gather_body(ids_hbm, wemb_hbm, out_hbm, idx_v, rows_a, rows_b,
                 gsem_a, gsem_b, wsem_a, wsem_b):
    c = lax.axis_index("c")
    s = lax.axis_index("s")
    wid = s * 2 + c
    base_w = wid * _TOK_W

    def pair_body(p, carry):
        g0 = 2 * p
        base0 = base_w + g0 * _C
        gm0 = 2 * lax.rem(p, 2)

        @pl.when(lax.rem(p, 2) == 0)
        def _():
            row0 = pl.multiple_of(base0 // 128, 8)
            pltpu.sync_copy(ids_hbm.at[pl.ds(row0, 8)], idx_v)

        def start(rows_v, gm, gsem):
            return [
                pltpu.async_copy(
                    wemb_hbm.at[idx_v.at[gm * 2 + j]],
                    rows_v.at[pl.ds(j * 128, 128)],
                    gsem,
                )
                for j in range(_C // 128)
            ]

        cps_a = start(rows_a, gm0, gsem_a)
        cps_b = start(rows_b, gm0 + 1, gsem_b)
        for cp in cps_a:
            cp.wait()
        wb_a = pltpu.async_copy(rows_a, out_hbm.at[pl.ds(base0, _C)], wsem_a)
        for cp in cps_b:
            cp.wait()
        wb_b = pltpu.async_copy(rows_b, out_hbm.at[pl.ds(base0 + _C, _C)],
                                wsem_b)
        wb_a.wait()
        wb_b.wait()
        return carry

    lax.fori_loop(0, _NCH // 2, pair_body, 0)


_sc_gather = pl.kernel(
    _gather_body,
    out_type=jax.ShapeDtypeStruct((_N, _D), jnp.float32),
    mesh=plsc.VectorSubcoreMesh(core_axis_name="c", subcore_axis_name="s"),
    compiler_params=pltpu.CompilerParams(needs_layout_passes=False),
    scratch_types=[
        pltpu.VMEM((8, 128), jnp.int32),
        pltpu.VMEM((_C, _D), jnp.float32),
        pltpu.VMEM((_C, _D), jnp.float32),
        pltpu.SemaphoreType.DMA,
        pltpu.SemaphoreType.DMA,
        pltpu.SemaphoreType.DMA,
        pltpu.SemaphoreType.DMA,
    ],
)


_SPS = 8
_BLK = _SPS * _L


def _ln_body(x_ref, ttf_ref, pos_ref, temb_ref, gam_ref, bet_ref, o_ref):
    x = x_ref[...]
    ttf = ttf_ref[...]
    t0 = temb_ref[0:1, :]
    t1 = temb_ref[1:2, :]
    x = x + pos_ref[...] + t0 + ttf * (t1 - t0)
    mean = jnp.mean(x, axis=-1, keepdims=True)
    xc = x - mean
    var = jnp.mean(xc * xc, axis=-1, keepdims=True)
    inv = lax.rsqrt(var + _EPS)
    o_ref[...] = xc * inv * gam_ref[...] + bet_ref[...]


_tc_ln = pl.pallas_call(
    _ln_body,
    grid=(_B // _SPS,),
    in_specs=[
        pl.BlockSpec((_BLK, _D), lambda i: (i, 0)),
        pl.BlockSpec((_BLK, 1), lambda i: (i, 0)),
        pl.BlockSpec((_BLK, _D), lambda i: (0, 0)),
        pl.BlockSpec((_T, _D), lambda i: (0, 0)),
        pl.BlockSpec((1, _D), lambda i: (0, 0)),
        pl.BlockSpec((1, _D), lambda i: (0, 0)),
    ],
    out_specs=pl.BlockSpec((_BLK, _D), lambda i: (i, 0)),
    out_shape=jax.ShapeDtypeStruct((_N, _D), jnp.float32),
)


def kernel(input_ids, token_type_ids, word_emb, pos_emb, type_emb,
           ln_gamma, ln_beta):
    ids = input_ids.reshape(-1).astype(jnp.int32).reshape(_N // 128, 128)
    ttf = token_type_ids.reshape(_N, 1).astype(jnp.float32)
    rows = _sc_gather(ids, word_emb.astype(jnp.float32))
    pos_t = jnp.tile(pos_emb.astype(jnp.float32), (_SPS, 1))
    out = _tc_ln(rows, ttf, pos_t, type_emb.astype(jnp.float32),
                 ln_gamma.astype(jnp.float32).reshape(1, _D),
                 ln_beta.astype(jnp.float32).reshape(1, _D))
    return out.reshape(_B, _L, _D)


# speedup vs baseline: 7.3861x; 1.0621x over previous
import functools

import jax
import jax.numpy as jnp
from jax import lax
from jax.experimental import pallas as pl
from jax.experimental.pallas import tpu as pltpu
from jax.experimental.pallas import tpu_sc as plsc

_B, _L, _V, _P, _T, _D = 1024, 512, 100000, 512, 2, 128
_N = _B * _L
_EPS = 1e-12

_NW = 32
_TOK_W = _N // _NW
_C = 256
_NCH = _TOK_W // _C


def _gather_body(ids_hbm, wemb_hbm, out_hbm, idx_v, rows_a, rows_b,
                 gsem_a, gsem_b, wsem_a, wsem_b):
    c = lax.axis_index("c")
    s = lax.axis_index("s")
    wid = s * 2 + c
    base_w = wid * _TOK_W

    def pair_body(p, carry):
        g0 = 2 * p
        base0 = base_w + g0 * _C
        gm0 = 2 * lax.rem(p, 2)

        @pl.when(lax.rem(p, 2) == 0)
        def _():
            row0 = pl.multiple_of(base0 // 128, 8)
            pltpu.sync_copy(ids_hbm.at[pl.ds(row0, 8)], idx_v)

        def start(rows_v, gm, gsem):
            return [
                pltpu.async_copy(
                    wemb_hbm.at[idx_v.at[gm * 2 + j]],
                    rows_v.at[pl.ds(j * 128, 128)],
                    gsem,
                )
                for j in range(_C // 128)
            ]

        cps_a = start(rows_a, gm0, gsem_a)
        cps_b = start(rows_b, gm0 + 1, gsem_b)
        for cp in cps_a:
            cp.wait()
        wb_a = pltpu.async_copy(rows_a, out_hbm.at[pl.ds(base0, _C)], wsem_a)
        for cp in cps_b:
            cp.wait()
        wb_b = pltpu.async_copy(rows_b, out_hbm.at[pl.ds(base0 + _C, _C)],
                                wsem_b)
        wb_a.wait()
        wb_b.wait()
        return carry

    lax.fori_loop(0, _NCH // 2, pair_body, 0)


_sc_gather = pl.kernel(
    _gather_body,
    out_type=jax.ShapeDtypeStruct((_N, _D), jnp.float32),
    mesh=plsc.VectorSubcoreMesh(core_axis_name="c", subcore_axis_name="s"),
    compiler_params=pltpu.CompilerParams(needs_layout_passes=False),
    scratch_types=[
        pltpu.VMEM((8, 128), jnp.int32),
        pltpu.VMEM((_C, _D), jnp.float32),
        pltpu.VMEM((_C, _D), jnp.float32),
        pltpu.SemaphoreType.DMA,
        pltpu.SemaphoreType.DMA,
        pltpu.SemaphoreType.DMA,
        pltpu.SemaphoreType.DMA,
    ],
)


_SPS = 16
_BLK = _SPS * _L


def _ln_body(x_ref, ttf_ref, pos_ref, temb_ref, gam_ref, bet_ref, o_ref):
    x = x_ref[...]
    ttf = ttf_ref[...]
    t0 = temb_ref[0:1, :]
    t1 = temb_ref[1:2, :]
    x = x + pos_ref[...] + t0 + ttf * (t1 - t0)
    mean = jnp.mean(x, axis=-1, keepdims=True)
    xc = x - mean
    var = jnp.mean(xc * xc, axis=-1, keepdims=True)
    inv = lax.rsqrt(var + _EPS)
    o_ref[...] = xc * inv * gam_ref[...] + bet_ref[...]


_tc_ln = pl.pallas_call(
    _ln_body,
    grid=(_B // _SPS,),
    in_specs=[
        pl.BlockSpec((_BLK, _D), lambda i: (i, 0)),
        pl.BlockSpec((_BLK, 1), lambda i: (i, 0)),
        pl.BlockSpec((_BLK, _D), lambda i: (0, 0)),
        pl.BlockSpec((_T, _D), lambda i: (0, 0)),
        pl.BlockSpec((1, _D), lambda i: (0, 0)),
        pl.BlockSpec((1, _D), lambda i: (0, 0)),
    ],
    out_specs=pl.BlockSpec((_BLK, _D), lambda i: (i, 0)),
    out_shape=jax.ShapeDtypeStruct((_N, _D), jnp.float32),
)


def kernel(input_ids, token_type_ids, word_emb, pos_emb, type_emb,
           ln_gamma, ln_beta):
    ids = input_ids.reshape(-1).astype(jnp.int32).reshape(_N // 128, 128)
    ttf = token_type_ids.reshape(_N, 1).astype(jnp.float32)
    rows = _sc_gather(ids, word_emb.astype(jnp.float32))
    pos_t = jnp.tile(pos_emb.astype(jnp.float32), (_SPS, 1))
    out = _tc_ln(rows, ttf, pos_t, type_emb.astype(jnp.float32),
                 ln_gamma.astype(jnp.float32).reshape(1, _D),
                 ln_beta.astype(jnp.float32).reshape(1, _D))
    return out.reshape(_B, _L, _D)
